# baseline (speedup 1.0000x reference)
import functools

import jax
import jax.numpy as jnp
from jax import lax
from jax.experimental import pallas as pl
from jax.experimental.pallas import tpu as pltpu
from jax.experimental.pallas import tpu_sc as plsc

N = 10000
E = 160000
R = 10240
NR = 2 * R
EP = 163840
D = 256
HALF = 128
EMB = 64
BLK = 512
GRID = NR // BLK
GBLK = R // BLK
NSUB = 16
EDGE_W = EP // NSUB
NBLK_E = EDGE_W // 128
SLICE = R // NSUB
ROWS_W = NR // NSUB

f32 = jnp.float32
i32 = jnp.int32


def _prep_body(utab, btab, ui, bi, dstp, ones_hbm, zeros128,
               u_out, b_out, deg_out,
               idx_v, rows_v, ones_v, deg_sh, sem):
    c = lax.axis_index("c")
    s = lax.axis_index("s")

    @pl.when(c == 1)
    def _emb():
        for blk in range(ROWS_W // 128):
            base = pl.multiple_of(s * ROWS_W + blk * 128, 128)
            pltpu.sync_copy(ui.at[pl.ds(base, 128)], idx_v)
            pltpu.async_copy(utab.at[idx_v], rows_v, sem).wait()
            pltpu.sync_copy(rows_v, u_out.at[pl.ds(base, 128)])
            pltpu.sync_copy(bi.at[pl.ds(base, 128)], idx_v)
            pltpu.async_copy(btab.at[idx_v], rows_v, sem).wait()
            pltpu.sync_copy(rows_v, b_out.at[pl.ds(base, 128)])

    @pl.when(c == 0)
    def _deg():
        sbase = pl.multiple_of(s * SLICE, 128)
        pltpu.sync_copy(zeros128, deg_sh.at[pl.ds(sbase, SLICE)])
        pltpu.sync_copy(ones_hbm, ones_v)
        plsc.subcore_barrier()

        def body(blk, carry):
            base = pl.multiple_of(s * EDGE_W + blk * 128, 128)
            pltpu.sync_copy(dstp.at[pl.ds(base, 128)], idx_v)
            pltpu.sync_copy(ones_v, deg_sh.at[idx_v], add=True)
            return carry

        lax.fori_loop(0, NBLK_E, body, 0)
        plsc.subcore_barrier()
        pltpu.sync_copy(deg_sh.at[pl.ds(sbase, SLICE)],
                        deg_out.at[pl.ds(sbase, SLICE)])


TILE_E = 64
NT = EDGE_W // TILE_E
NSLOT = 4
LAG = 2


def _spmm_body(hsa, hsb, srcp, dst3, zeros128,
               scata, scatb,
               src_big, dst_ring, rows_v, out_sh, *sems):
    gsems, ssems, isems = sems[0:4], sems[4:8], sems[8:12]
    ch = lax.axis_index("c")
    s = lax.axis_index("s")
    sbase = pl.multiple_of(s * SLICE, 128)
    ebase = pl.multiple_of(s * EDGE_W, 128)
    pltpu.sync_copy(srcp.at[pl.ds(ebase, EDGE_W)], src_big)

    def _run_half(hs_ref, scat_ref):
        pltpu.sync_copy(zeros128, out_sh.at[pl.ds(sbase, SLICE)])

        def fire(t, j, hs_ref=hs_ref):
            pltpu.async_copy(dst3.at[s * NT + t], dst_ring.at[j], isems[j])
            eoff = pl.multiple_of(t * TILE_E, 8)
            pltpu.async_copy(
                hs_ref.at[src_big.at[pl.ds(eoff, TILE_E)]],
                rows_v.at[pl.ds(j * TILE_E, TILE_E)], gsems[j])

        def drain_g(j, hs_ref=hs_ref):
            pltpu.make_async_copy(hs_ref.at[pl.ds(0, TILE_E)],
                                  rows_v.at[pl.ds(j * TILE_E, TILE_E)],
                                  gsems[j]).wait()
            pltpu.make_async_copy(dst3.at[0], dst_ring.at[j],
                                  isems[j]).wait()

        def scat(j):
            pltpu.async_copy(rows_v.at[pl.ds(j * TILE_E, TILE_E)],
                             out_sh.at[dst_ring.at[j]], ssems[j], add=True)

        def drain_s(j, hs_ref=hs_ref):
            pltpu.make_async_copy(hs_ref.at[pl.ds(0, TILE_E)],
                                  rows_v.at[pl.ds(j * TILE_E, TILE_E)],
                                  ssems[j]).wait()

        fire(0, 0)
        fire(1, 1)
        plsc.subcore_barrier()

        def body(u, carry):
            for j in range(NSLOT):
                t = NSLOT * u + j

                @pl.when(t - LAG >= 0)
                def _():
                    drain_s((j + NSLOT - LAG) % NSLOT)

                @pl.when(t + LAG < NT)
                def _():
                    fire(t + LAG, (j + LAG) % NSLOT)

                drain_g(j)
                scat(j)
            return carry

        lax.fori_loop(0, NT // NSLOT, body, 0)
        drain_s((NT - 2) % NSLOT)
        drain_s((NT - 1) % NSLOT)
        plsc.subcore_barrier()
        pltpu.sync_copy(out_sh.at[pl.ds(sbase, SLICE)],
                        scat_ref.at[pl.ds(sbase, SLICE)])

    @pl.when(ch == 0)
    def _():
        _run_half(hsa, scata)

    @pl.when(ch == 1)
    def _():
        _run_half(hsb, scatb)


def _sc_mesh():
    return plsc.VectorSubcoreMesh(core_axis_name="c", subcore_axis_name="s")


def _dinv(deg_ref, i):
    d = deg_ref[:, 0:1] + 1.0
    rows = lax.broadcasted_iota(i32, (BLK, 1), 0) + i * BLK
    return jnp.where(rows < N, lax.rsqrt(d), 0.0)


def _mm1_body(xn_ref, u_ref, b_ref, wa_ref, we_ref, deg_ref,
              hsa_ref, hsb_ref):
    i = pl.program_id(0)
    dinv = _dinv(deg_ref, i)
    emb = u_ref[...] + b_ref[...]
    h = (jnp.dot(xn_ref[...], wa_ref[...], preferred_element_type=f32, precision=lax.Precision.HIGHEST)
         + jnp.dot(emb, we_ref[...], preferred_element_type=f32, precision=lax.Precision.HIGHEST)) * dinv
    hsa_ref[...] = h[:, :HALF]
    hsb_ref[...] = h[:, HALF:]


def _mm2_body(hsa_ref, hsb_ref, sca_ref, scb_ref, deg_ref, b1_ref, w_ref,
              oa_ref, ob_ref):
    i = pl.program_id(0)
    dinv = _dinv(deg_ref, i)
    hs = jnp.concatenate([hsa_ref[...], hsb_ref[...]], axis=1)
    sc = jnp.concatenate([sca_ref[...], scb_ref[...]], axis=1)
    t = jnp.maximum(dinv * (hs + sc) + b1_ref[...], 0.0)
    h2 = jnp.dot(t, w_ref[...], preferred_element_type=f32, precision=lax.Precision.HIGHEST) * dinv
    oa_ref[...] = h2[:, :HALF]
    ob_ref[...] = h2[:, HALF:]


def _mm3_body(hsa_ref, hsb_ref, sca_ref, scb_ref, deg_ref, b2_ref,
              g_ref, s1_ref):
    i = pl.program_id(0)
    dinv = _dinv(deg_ref, i)
    hs = jnp.concatenate([hsa_ref[...], hsb_ref[...]], axis=1)
    sc = jnp.concatenate([sca_ref[...], scb_ref[...]], axis=1)
    gval = jnp.maximum(dinv * (hs + sc) + b2_ref[...], 0.0)
    g_ref[...] = gval
    rows_g = lax.broadcasted_iota(i32, (BLK, 1), 0) + i * BLK
    contrib = jnp.sum(jnp.where(rows_g < N, gval, 0.0), axis=0, keepdims=True)

    @pl.when(i == 0)
    def _():
        s1_ref[...] = jnp.zeros_like(s1_ref)

    s1_ref[...] += contrib


def _read_body(g_ref, s1_ref, wd_ref, bd_ref, out_ref):
    c = jax.nn.sigmoid(s1_ref[...] * (1.0 / N))
    w = lax.dot_general(wd_ref[...], c, (((1,), (1,)), ((), ())),
                        preferred_element_type=f32, precision=lax.Precision.HIGHEST)
    out_ref[...] = (jnp.dot(g_ref[...], w, preferred_element_type=f32, precision=lax.Precision.HIGHEST)
                    + bd_ref[:, :1])


def _full(shape):
    return pl.BlockSpec(shape, lambda i: (0, 0))


def _rows(shape):
    return pl.BlockSpec(shape, lambda i: (i, 0))


def _grows(g_idx):
    return pl.BlockSpec((BLK, HALF), lambda i, g=g_idx: (i + g * GBLK, 0))


def kernel(x_num, uniprot_idx, bin_idx, shuf_num, shuf_uniprot, shuf_bin,
           edge_index, uniprot_table, bin_table, W1, b1, W2, b2, Wd, bd):
    src = edge_index[0]
    dst = edge_index[1]
    pe = EP - E
    srcp = jnp.concatenate([src, jnp.full((pe,), N, i32)])
    dstp = jnp.concatenate([dst, jnp.full((pe,), N, i32)])
    pad_n = R - N
    ui2 = jnp.concatenate([uniprot_idx, jnp.zeros((pad_n,), i32),
                           shuf_uniprot, jnp.zeros((pad_n,), i32)])
    bi2 = jnp.concatenate([bin_idx, jnp.zeros((pad_n,), i32),
                           shuf_bin, jnp.zeros((pad_n,), i32)])
    xn2 = jnp.concatenate([jnp.pad(x_num, ((0, pad_n), (0, 0))),
                           jnp.pad(shuf_num, ((0, pad_n), (0, 0)))], axis=0)
    ones128 = jnp.ones((128, HALF), f32)
    zeros128 = jnp.zeros((SLICE, HALF), f32)
    utab_p = jnp.pad(uniprot_table, ((0, 0), (0, HALF - EMB)))
    btab_p = jnp.pad(bin_table, ((0, 0), (HALF - EMB, 0)))

    prep = pl.kernel(
        _prep_body,
        out_type=[jax.ShapeDtypeStruct((NR, HALF), f32),
                  jax.ShapeDtypeStruct((NR, HALF), f32),
                  jax.ShapeDtypeStruct((R, HALF), f32)],
        mesh=_sc_mesh(),
        scratch_types=[pltpu.VMEM((128,), i32),
                       pltpu.VMEM((128, HALF), f32),
                       pltpu.VMEM((128, HALF), f32),
                       pltpu.VMEM_SHARED((R, HALF), f32),
                       pltpu.SemaphoreType.DMA],
    )
    u_out, b_out, deg = prep(utab_p, btab_p, ui2, bi2, dstp,
                             ones128, zeros128)

    spmm = pl.kernel(
        _spmm_body,
        out_type=[jax.ShapeDtypeStruct((R, HALF), f32),
                  jax.ShapeDtypeStruct((R, HALF), f32)],
        mesh=_sc_mesh(),
        scratch_types=[pltpu.VMEM((EDGE_W,), i32),
                       pltpu.VMEM((NSLOT, TILE_E), i32),
                       pltpu.VMEM((NSLOT * TILE_E, HALF), f32),
                       pltpu.VMEM_SHARED((R, HALF), f32)]
                      + [pltpu.SemaphoreType.DMA] * 12,
    )

    dst3 = dstp.reshape(EP // TILE_E, TILE_E)
    b1r = b1.reshape(1, D)
    b2r = b2.reshape(1, D)
    bdr = jnp.full((1, HALF), bd, f32)
    W1a, W1e = W1[:HALF], W1[HALF:]

    gs, s1 = [], None
    for g_idx in (0, 1):
        hsa, hsb = pl.pallas_call(
            _mm1_body,
            grid=(GBLK,),
            in_specs=[_grows(g_idx)] * 3 + [_full((HALF, D)),
                                            _full((HALF, D)),
                                            _rows((BLK, HALF))],
            out_specs=[_rows((BLK, HALF)), _rows((BLK, HALF))],
            out_shape=[jax.ShapeDtypeStruct((R, HALF), f32)] * 2,
        )(xn2, u_out, b_out, W1a, W1e, deg)

        sca, scb = spmm(hsa, hsb, srcp, dst3, zeros128)

        h2a, h2b = pl.pallas_call(
            _mm2_body,
            grid=(GBLK,),
            in_specs=[_rows((BLK, HALF))] * 5 + [_full((1, D)),
                                                 _full((D, D))],
            out_specs=[_rows((BLK, HALF)), _rows((BLK, HALF))],
            out_shape=[jax.ShapeDtypeStruct((R, HALF), f32)] * 2,
        )(hsa, hsb, sca, scb, deg, b1r, W2)

        s2a, s2b = spmm(h2a, h2b, srcp, dst3, zeros128)

        G, s1g = pl.pallas_call(
            _mm3_body,
            grid=(GBLK,),
            in_specs=[_rows((BLK, HALF))] * 5 + [_full((1, D))],
            out_specs=[_rows((BLK, D)), pl.BlockSpec((1, D), lambda i: (0, 0))],
            out_shape=[jax.ShapeDtypeStruct((R, D), f32),
                       jax.ShapeDtypeStruct((1, D), f32)],
        )(h2a, h2b, s2a, s2b, deg, b2r)
        if g_idx == 0:
            s1 = s1g
        gs.append(G)

    parts = []
    for G in gs:
        o = pl.pallas_call(
            _read_body,
            grid=(GBLK,),
            in_specs=[_rows((BLK, D)), _full((1, D)), _full((D, D)),
                      _full((1, HALF))],
            out_specs=pl.BlockSpec((BLK, 1), lambda i: (i, 0)),
            out_shape=jax.ShapeDtypeStruct((R, 1), f32),
        )(G, s1, Wd, bdr)
        parts.append(o[:N, 0])
    return jnp.concatenate(parts)


# speedup vs baseline: 1.0148x; 1.0148x over previous
import functools

import jax
import jax.numpy as jnp
from jax import lax
from jax.experimental import pallas as pl
from jax.experimental.pallas import tpu as pltpu
from jax.experimental.pallas import tpu_sc as plsc

N = 10000
E = 160000
R = 10240
NR = 2 * R
EP = 163840
D = 256
HALF = 128
EMB = 64
BLK = 512
GRID = NR // BLK
GBLK = R // BLK
NSUB = 16
EDGE_W = EP // NSUB
NBLK_E = EDGE_W // 128
SLICE = R // NSUB
ROWS_W = NR // NSUB

f32 = jnp.float32
bf16 = jnp.bfloat16
i32 = jnp.int32


def _prep_body(utab, btab, ui, bi, dstp, ones_hbm, zeros128,
               u_out, b_out, deg_out,
               idx_v, rows_v, ones_v, deg_sh, sem):
    c = lax.axis_index("c")
    s = lax.axis_index("s")

    @pl.when(c == 1)
    def _emb():
        for blk in range(ROWS_W // 128):
            base = pl.multiple_of(s * ROWS_W + blk * 128, 128)
            pltpu.sync_copy(ui.at[pl.ds(base, 128)], idx_v)
            pltpu.async_copy(utab.at[idx_v], rows_v, sem).wait()
            pltpu.sync_copy(rows_v, u_out.at[pl.ds(base, 128)])
            pltpu.sync_copy(bi.at[pl.ds(base, 128)], idx_v)
            pltpu.async_copy(btab.at[idx_v], rows_v, sem).wait()
            pltpu.sync_copy(rows_v, b_out.at[pl.ds(base, 128)])

    @pl.when(c == 0)
    def _deg():
        sbase = pl.multiple_of(s * SLICE, 128)
        pltpu.sync_copy(zeros128, deg_sh.at[pl.ds(sbase, SLICE)])
        pltpu.sync_copy(ones_hbm, ones_v)
        plsc.subcore_barrier()

        def body(blk, carry):
            base = pl.multiple_of(s * EDGE_W + blk * 128, 128)
            pltpu.sync_copy(dstp.at[pl.ds(base, 128)], idx_v)
            pltpu.sync_copy(ones_v, deg_sh.at[idx_v], add=True)
            return carry

        lax.fori_loop(0, NBLK_E, body, 0)
        plsc.subcore_barrier()
        pltpu.sync_copy(deg_sh.at[pl.ds(sbase, SLICE)],
                        deg_out.at[pl.ds(sbase, SLICE)])


TILE_E = 64
NT = EDGE_W // TILE_E
NSLOT = 4
LAG = 2


def _spmm_body(hsa, hsb, src2, dst3, zeros128,
               scata, scatb,
               src_big, dst_ring, rows_v, out_sh, *sems):
    gsems, ssems, isems = sems[0:4], sems[4:8], sems[8:12]
    g = lax.axis_index("c")
    s = lax.axis_index("s")
    sbase = pl.multiple_of(s * SLICE, 128)
    ebase = pl.multiple_of(g * EP + s * EDGE_W, 128)
    pltpu.sync_copy(src2.at[pl.ds(ebase, EDGE_W)], src_big)

    for hs_ref, scat_ref in ((hsa, scata), (hsb, scatb)):
        pltpu.sync_copy(zeros128, out_sh.at[pl.ds(sbase, SLICE)])

        def fire(t, j, hs_ref=hs_ref):
            pltpu.async_copy(dst3.at[s * NT + t], dst_ring.at[j], isems[j])
            eoff = pl.multiple_of(t * TILE_E, 8)
            pltpu.async_copy(
                hs_ref.at[src_big.at[pl.ds(eoff, TILE_E)]],
                rows_v.at[pl.ds(j * TILE_E, TILE_E)], gsems[j])

        def drain_g(j, hs_ref=hs_ref):
            pltpu.make_async_copy(hs_ref.at[pl.ds(0, TILE_E)],
                                  rows_v.at[pl.ds(j * TILE_E, TILE_E)],
                                  gsems[j]).wait()
            pltpu.make_async_copy(dst3.at[0], dst_ring.at[j],
                                  isems[j]).wait()

        def scat(j):
            pltpu.async_copy(rows_v.at[pl.ds(j * TILE_E, TILE_E)],
                             out_sh.at[dst_ring.at[j]], ssems[j], add=True)

        def drain_s(j, hs_ref=hs_ref):
            pltpu.make_async_copy(hs_ref.at[pl.ds(0, TILE_E)],
                                  rows_v.at[pl.ds(j * TILE_E, TILE_E)],
                                  ssems[j]).wait()

        fire(0, 0)
        fire(1, 1)
        plsc.subcore_barrier()

        def body(u, carry):
            for j in range(NSLOT):
                t = NSLOT * u + j

                @pl.when(t - LAG >= 0)
                def _():
                    drain_s((j + NSLOT - LAG) % NSLOT)

                @pl.when(t + LAG < NT)
                def _():
                    fire(t + LAG, (j + LAG) % NSLOT)

                drain_g(j)
                scat(j)
            return carry

        lax.fori_loop(0, NT // NSLOT, body, 0)
        drain_s((NT - 2) % NSLOT)
        drain_s((NT - 1) % NSLOT)
        plsc.subcore_barrier()
        obase = pl.multiple_of(g * R + sbase, 128)
        pltpu.sync_copy(out_sh.at[pl.ds(sbase, SLICE)],
                        scat_ref.at[pl.ds(obase, SLICE)])


def _sc_mesh():
    return plsc.VectorSubcoreMesh(core_axis_name="c", subcore_axis_name="s")


def _dot3(a, b):
    ah = a.astype(bf16)
    al = (a - ah.astype(f32)).astype(bf16)
    bh = b.astype(bf16)
    bl = (b - bh.astype(f32)).astype(bf16)
    return (jnp.dot(ah, bh, preferred_element_type=f32)
            + jnp.dot(ah, bl, preferred_element_type=f32)
            + jnp.dot(al, bh, preferred_element_type=f32))


def _dinv(deg_ref, i):
    d = deg_ref[:, 0:1] + 1.0
    rows = lax.broadcasted_iota(i32, (BLK, 1), 0) + lax.rem(i, GBLK) * BLK
    return jnp.where(rows < N, 1.0 / jnp.sqrt(d), 0.0)


def _mm1_body(xn_ref, u_ref, b_ref, wa_ref, we_ref, deg_ref,
              hsa_ref, hsb_ref):
    i = pl.program_id(0)
    dinv = _dinv(deg_ref, i)
    emb = u_ref[...] + b_ref[...]
    h = (_dot3(xn_ref[...], wa_ref[...]) + _dot3(emb, we_ref[...])) * dinv
    hsa_ref[...] = h[:, :HALF]
    hsb_ref[...] = h[:, HALF:]


def _mm2_body(hsa_ref, hsb_ref, sca_ref, scb_ref, deg_ref, b1_ref, w_ref,
              oa_ref, ob_ref):
    i = pl.program_id(0)
    dinv = _dinv(deg_ref, i)
    hs = jnp.concatenate([hsa_ref[...], hsb_ref[...]], axis=1)
    sc = jnp.concatenate([sca_ref[...], scb_ref[...]], axis=1)
    t = jnp.maximum(dinv * (hs + sc) + b1_ref[...], 0.0)
    h2 = _dot3(t, w_ref[...]) * dinv
    oa_ref[...] = h2[:, :HALF]
    ob_ref[...] = h2[:, HALF:]


def _mm3_body(hsa_ref, hsb_ref, sca_ref, scb_ref, deg_ref, b2_ref,
              g_ref, s1_ref):
    i = pl.program_id(0)
    dinv = _dinv(deg_ref, i)
    hs = jnp.concatenate([hsa_ref[...], hsb_ref[...]], axis=1)
    sc = jnp.concatenate([sca_ref[...], scb_ref[...]], axis=1)
    gval = jnp.maximum(dinv * (hs + sc) + b2_ref[...], 0.0)
    g_ref[...] = gval
    rows_g = lax.broadcasted_iota(i32, (BLK, 1), 0) + i * BLK
    contrib = jnp.sum(jnp.where(rows_g < N, gval, 0.0), axis=0, keepdims=True)

    @pl.when(i == 0)
    def _():
        s1_ref[...] = jnp.zeros_like(s1_ref)

    s1_ref[...] += contrib


def _read_body(g_ref, s1_ref, wd_ref, bd_ref, out_ref):
    c = jax.nn.sigmoid(s1_ref[...] * (1.0 / N))
    p = _dot3(g_ref[...], wd_ref[...])
    out_ref[...] = jnp.sum(p * c, axis=1, keepdims=True) + bd_ref[:, :1]


def _full(shape):
    return pl.BlockSpec(shape, lambda i: (0, 0))


def _rows(shape):
    return pl.BlockSpec(shape, lambda i: (i, 0))


def _degspec():
    return pl.BlockSpec((BLK, HALF), lambda i: (lax.rem(i, GBLK), 0))


def kernel(x_num, uniprot_idx, bin_idx, shuf_num, shuf_uniprot, shuf_bin,
           edge_index, uniprot_table, bin_table, W1, b1, W2, b2, Wd, bd):
    src = edge_index[0]
    dst = edge_index[1]
    pe = EP - E
    srcp = jnp.concatenate([src, jnp.full((pe,), N, i32)])
    dstp = jnp.concatenate([dst, jnp.full((pe,), N, i32)])
    src2 = jnp.concatenate([srcp, srcp + R])
    pad_n = R - N
    ui2 = jnp.concatenate([uniprot_idx, jnp.zeros((pad_n,), i32),
                           shuf_uniprot, jnp.zeros((pad_n,), i32)])
    bi2 = jnp.concatenate([bin_idx, jnp.zeros((pad_n,), i32),
                           shuf_bin, jnp.zeros((pad_n,), i32)])
    xn2 = jnp.concatenate([jnp.pad(x_num, ((0, pad_n), (0, 0))),
                           jnp.pad(shuf_num, ((0, pad_n), (0, 0)))], axis=0)
    ones128 = jnp.ones((128, HALF), f32)
    zeros128 = jnp.zeros((SLICE, HALF), f32)
    utab_p = jnp.pad(uniprot_table, ((0, 0), (0, HALF - EMB)))
    btab_p = jnp.pad(bin_table, ((0, 0), (HALF - EMB, 0)))

    prep = pl.kernel(
        _prep_body,
        out_type=[jax.ShapeDtypeStruct((NR, HALF), f32),
                  jax.ShapeDtypeStruct((NR, HALF), f32),
                  jax.ShapeDtypeStruct((R, HALF), f32)],
        mesh=_sc_mesh(),
        scratch_types=[pltpu.VMEM((128,), i32),
                       pltpu.VMEM((128, HALF), f32),
                       pltpu.VMEM((128, HALF), f32),
                       pltpu.VMEM_SHARED((R, HALF), f32),
                       pltpu.SemaphoreType.DMA],
    )
    u_out, b_out, deg = prep(utab_p, btab_p, ui2, bi2, dstp,
                             ones128, zeros128)

    spmm = pl.kernel(
        _spmm_body,
        out_type=[jax.ShapeDtypeStruct((NR, HALF), f32),
                  jax.ShapeDtypeStruct((NR, HALF), f32)],
        mesh=_sc_mesh(),
        scratch_types=[pltpu.VMEM((EDGE_W,), i32),
                       pltpu.VMEM((NSLOT, TILE_E), i32),
                       pltpu.VMEM((NSLOT * TILE_E, HALF), f32),
                       pltpu.VMEM_SHARED((R, HALF), f32)]
                      + [pltpu.SemaphoreType.DMA] * 12,
    )

    hsa, hsb = pl.pallas_call(
        _mm1_body,
        grid=(GRID,),
        in_specs=[_rows((BLK, HALF))] * 3 + [_full((HALF, D)),
                                             _full((HALF, D)), _degspec()],
        out_specs=[_rows((BLK, HALF)), _rows((BLK, HALF))],
        out_shape=[jax.ShapeDtypeStruct((NR, HALF), f32)] * 2,
    )(xn2, u_out, b_out, W1[:HALF], W1[HALF:], deg)

    dst3 = dstp.reshape(EP // TILE_E, TILE_E)
    sca, scb = spmm(hsa, hsb, src2, dst3, zeros128)

    h2a, h2b = pl.pallas_call(
        _mm2_body,
        grid=(GRID,),
        in_specs=[_rows((BLK, HALF))] * 4 + [_degspec(), _full((1, D)),
                                             _full((D, D))],
        out_specs=[_rows((BLK, HALF)), _rows((BLK, HALF))],
        out_shape=[jax.ShapeDtypeStruct((NR, HALF), f32)] * 2,
    )(hsa, hsb, sca, scb, deg, b1.reshape(1, D), W2)

    s2a, s2b = spmm(h2a, h2b, src2, dst3, zeros128)

    G, s1 = pl.pallas_call(
        _mm3_body,
        grid=(GRID,),
        in_specs=[_rows((BLK, HALF))] * 4 + [_degspec(), _full((1, D))],
        out_specs=[_rows((BLK, D)), pl.BlockSpec((1, D), lambda i: (0, 0))],
        out_shape=[jax.ShapeDtypeStruct((NR, D), f32),
                   jax.ShapeDtypeStruct((1, D), f32)],
    )(h2a, h2b, s2a, s2b, deg, b2.reshape(1, D))

    out = pl.pallas_call(
        _read_body,
        grid=(GRID,),
        in_specs=[_rows((BLK, D)), _full((1, D)), _full((D, D)),
                  _full((1, HALF))],
        out_specs=pl.BlockSpec((BLK, 1), lambda i: (i, 0)),
        out_shape=jax.ShapeDtypeStruct((NR, 1), f32),
    )(G, s1, Wd, jnp.full((1, HALF), bd, f32))

    lo = out[:, 0]
    return jnp.concatenate([lo[:N], lo[R:R + N]])
